# baseline (speedup 1.0000x reference)
import functools

import jax
import jax.numpy as jnp
from jax import lax
from jax.experimental import pallas as pl
from jax.experimental.pallas import tpu as pltpu
from jax.experimental.pallas import tpu_sc as plsc

NUM_ROWS = 1000000
DIM = 64
BATCH = 4096
SEQ = 50

_info = plsc.get_sparse_core_info()
NC, NS, NL = _info.num_cores, _info.num_subcores, _info.num_lanes
NW = NC * NS

CHUNK = 128
GROUPS = CHUNK // NL

_mesh = plsc.VectorSubcoreMesh(core_axis_name="c", subcore_axis_name="s")


@functools.partial(
    pl.kernel,
    out_type=jax.ShapeDtypeStruct((SEQ, DIM, BATCH), jnp.float32),
    mesh=_mesh,
    scratch_types=[
        pltpu.VMEM((SEQ, CHUNK), jnp.int32),
        pltpu.VMEM((2, CHUNK), jnp.int32),
        pltpu.VMEM((2, CHUNK), jnp.int32),
        pltpu.VMEM((2, CHUNK, CHUNK), jnp.float32),
        pltpu.VMEM((2, DIM, CHUNK), jnp.float32),
        pltpu.SemaphoreType.DMA((2,)),
        pltpu.SemaphoreType.DMA((2,)),
    ],
    compiler_params=pltpu.CompilerParams(
        use_tc_tiling_on_sc=True, needs_layout_passes=False
    ),
)
def _emb_kernel(idx_hbm, table_hbm, out_hbm, idx_v, pair_v, colb_v, bufs, slabs,
                gsem, wsem):
    wid = lax.axis_index("s") * NC + lax.axis_index("c")
    b0 = wid * CHUNK
    pltpu.sync_copy(idx_hbm.at[:, pl.ds(b0, CHUNK)], idx_v)

    def prep(s, slot):
        for g in range(GROUPS):
            raw = idx_v[s, pl.ds(g * NL, NL)]
            pair_v[slot, pl.ds(g * NL, NL)] = lax.shift_right_logical(raw, 1)
            colb_v[slot, pl.ds(g * NL, NL)] = (raw & 1) * DIM

    def start_gather(slot):
        pltpu.async_copy(table_hbm.at[pair_v.at[slot]], bufs.at[slot],
                         gsem.at[slot])

    def wait_gather(slot):
        pltpu.make_async_copy(table_hbm.at[pair_v.at[slot]], bufs.at[slot],
                              gsem.at[slot]).wait()

    def start_write(s, slot):
        pltpu.async_copy(slabs.at[slot],
                         out_hbm.at[s, :, pl.ds(b0, CHUNK)], wsem.at[slot])

    def wait_write(s, slot):
        pltpu.make_async_copy(slabs.at[slot],
                              out_hbm.at[s, :, pl.ds(b0, CHUNK)],
                              wsem.at[slot]).wait()

    def transpose_select(slot):
        for g in range(GROUPS):
            rows = jax.lax.broadcasted_iota(jnp.int32, (NL,), 0) + g * NL
            colb = colb_v[slot, pl.ds(g * NL, NL)]

            @pl.loop(0, DIM)
            def _c(c):
                vals = plsc.load_gather(bufs.at[slot], [rows, colb + c])
                slabs[slot, c, pl.ds(g * NL, NL)] = vals

    prep(0, 0)
    start_gather(0)

    @pl.loop(0, SEQ, step=2)
    def _grp(s0):
        for inner in range(2):
            s = s0 + inner
            slot = inner
            nxt = 1 - inner

            @pl.when(s + 1 < SEQ)
            def _():
                prep(s + 1, nxt)
                start_gather(nxt)

            wait_gather(slot)

            @pl.when(s >= 2)
            def _():
                wait_write(s - 2, slot)

            transpose_select(slot)
            start_write(s, slot)

    wait_write(SEQ - 2, 0)
    wait_write(SEQ - 1, 1)


def kernel(input, weight):
    idx_t = input.T
    table = weight.reshape(NUM_ROWS // 2, 2 * DIM)
    out = _emb_kernel(idx_t, table)
    return out.transpose(2, 0, 1)


# speedup vs baseline: 1.0029x; 1.0029x over previous
import functools

import jax
import jax.numpy as jnp
from jax import lax
from jax.experimental import pallas as pl
from jax.experimental.pallas import tpu as pltpu
from jax.experimental.pallas import tpu_sc as plsc

NUM_ROWS = 1000000
DIM = 64
BATCH = 4096
SEQ = 50

_info = plsc.get_sparse_core_info()
NC, NS, NL = _info.num_cores, _info.num_subcores, _info.num_lanes
NW = NC * NS

CHUNK = 128
GROUPS = CHUNK // NL

_mesh = plsc.VectorSubcoreMesh(core_axis_name="c", subcore_axis_name="s")


@functools.partial(
    pl.kernel,
    out_type=jax.ShapeDtypeStruct((SEQ, DIM, BATCH), jnp.float32),
    mesh=_mesh,
    scratch_types=[
        pltpu.VMEM((SEQ, CHUNK), jnp.int32),
        pltpu.VMEM((2, CHUNK), jnp.int32),
        pltpu.VMEM((2, CHUNK), jnp.int32),
        pltpu.VMEM((2, CHUNK, CHUNK), jnp.float32),
        pltpu.VMEM((2, DIM, CHUNK), jnp.float32),
        pltpu.SemaphoreType.DMA((2,)),
        pltpu.SemaphoreType.DMA((2,)),
    ],
    compiler_params=pltpu.CompilerParams(needs_layout_passes=False),
)
def _emb_kernel(idx_hbm, table_hbm, out_hbm, idx_v, pair_v, colb_v, bufs, slabs,
                gsem, wsem):
    wid = lax.axis_index("s") * NC + lax.axis_index("c")
    b0 = wid * CHUNK
    pltpu.sync_copy(idx_hbm.at[:, pl.ds(b0, CHUNK)], idx_v)

    def prep(s, slot):
        for g in range(GROUPS):
            raw = idx_v[s, pl.ds(g * NL, NL)]
            pair_v[slot, pl.ds(g * NL, NL)] = lax.shift_right_logical(raw, 1)
            colb_v[slot, pl.ds(g * NL, NL)] = (raw & 1) * DIM

    def start_gather(slot):
        pltpu.async_copy(table_hbm.at[pair_v.at[slot]], bufs.at[slot],
                         gsem.at[slot])

    def wait_gather(slot):
        pltpu.make_async_copy(table_hbm.at[pair_v.at[slot]], bufs.at[slot],
                              gsem.at[slot]).wait()

    def start_write(s, slot):
        pltpu.async_copy(slabs.at[slot],
                         out_hbm.at[s, :, pl.ds(b0, CHUNK)], wsem.at[slot])

    def wait_write(s, slot):
        pltpu.make_async_copy(slabs.at[slot],
                              out_hbm.at[s, :, pl.ds(b0, CHUNK)],
                              wsem.at[slot]).wait()

    def transpose_select(slot):
        iota = jax.lax.broadcasted_iota(jnp.int32, (NL,), 0)
        rows_l = [iota + g * NL for g in range(GROUPS)]
        colb_l = [colb_v[slot, pl.ds(g * NL, NL)] for g in range(GROUPS)]

        @pl.loop(0, DIM, unroll=4)
        def _c(c):
            for g in range(GROUPS):
                vals = plsc.load_gather(
                    bufs.at[slot], [rows_l[g], colb_l[g] + c]
                )
                slabs[slot, c, pl.ds(g * NL, NL)] = vals

    prep(0, 0)
    start_gather(0)

    @pl.loop(0, SEQ, step=2)
    def _grp(s0):
        for inner in range(2):
            s = s0 + inner
            slot = inner
            nxt = 1 - inner

            @pl.when(s + 1 < SEQ)
            def _():
                prep(s + 1, nxt)
                start_gather(nxt)

            wait_gather(slot)

            @pl.when(s >= 2)
            def _():
                wait_write(s - 2, slot)

            transpose_select(slot)
            start_write(s, slot)

    wait_write(SEQ - 2, 0)
    wait_write(SEQ - 1, 1)


def kernel(input, weight):
    idx_t = input.T
    table = weight.reshape(NUM_ROWS // 2, 2 * DIM)
    out = _emb_kernel(idx_t, table)
    return out.transpose(2, 0, 1)


# speedup vs baseline: 1.2874x; 1.2836x over previous
import functools

import jax
import jax.numpy as jnp
from jax import lax
from jax.experimental import pallas as pl
from jax.experimental.pallas import tpu as pltpu
from jax.experimental.pallas import tpu_sc as plsc

NUM_ROWS = 1000000
DIM = 64
BATCH = 4096
SEQ = 50

_info = plsc.get_sparse_core_info()
NC, NS, NL = _info.num_cores, _info.num_subcores, _info.num_lanes
NW = NC * NS

CHUNK = 128
GROUPS = CHUNK // NL

_mesh = plsc.VectorSubcoreMesh(core_axis_name="c", subcore_axis_name="s")


@functools.partial(
    pl.kernel,
    out_type=jax.ShapeDtypeStruct((SEQ, DIM, BATCH), jnp.float32),
    mesh=_mesh,
    scratch_types=[
        pltpu.VMEM((SEQ, CHUNK), jnp.int32),
        pltpu.VMEM((2, CHUNK), jnp.int32),
        pltpu.VMEM((2, CHUNK), jnp.int32),
        pltpu.VMEM((2, CHUNK, CHUNK), jnp.float32),
        pltpu.VMEM((2, DIM, CHUNK), jnp.float32),
        pltpu.SemaphoreType.DMA((2,)),
        pltpu.SemaphoreType.DMA((2,)),
    ],
    compiler_params=pltpu.CompilerParams(needs_layout_passes=False),
)
def _emb_kernel(idx_hbm, table_hbm, out_hbm, idx_v, pair_v, colb_v, bufs, slabs,
                gsem, wsem):
    wid = lax.axis_index("s") * NC + lax.axis_index("c")
    b0 = wid * CHUNK
    pltpu.sync_copy(idx_hbm.at[:, pl.ds(b0, CHUNK)], idx_v)

    def prep(s, slot):
        for g in range(GROUPS):
            raw = idx_v[s, pl.ds(g * NL, NL)]
            pair_v[slot, pl.ds(g * NL, NL)] = lax.shift_right_logical(raw, 1)
            colb_v[slot, pl.ds(g * NL, NL)] = (raw & 1) * DIM

    def start_gather(slot):
        pltpu.async_copy(table_hbm.at[pair_v.at[slot]], bufs.at[slot],
                         gsem.at[slot])

    def wait_gather(slot):
        pltpu.make_async_copy(table_hbm.at[pair_v.at[slot]], bufs.at[slot],
                              gsem.at[slot]).wait()

    def start_write(s, slot):
        pltpu.async_copy(slabs.at[slot],
                         out_hbm.at[s, :, pl.ds(b0, CHUNK)], wsem.at[slot])

    def wait_write(s, slot):
        pltpu.make_async_copy(slabs.at[slot],
                              out_hbm.at[s, :, pl.ds(b0, CHUNK)],
                              wsem.at[slot]).wait()

    def transpose_select(slot):
        iota = jax.lax.broadcasted_iota(jnp.int32, (NL,), 0)
        rows_l = [iota + g * NL for g in range(GROUPS)]
        colb_l = [colb_v[slot, pl.ds(g * NL, NL)] for g in range(GROUPS)]

        @pl.loop(0, DIM, unroll=4)
        def _c(c):
            cperm = (iota + c) & (DIM - 1)
            for g in range(GROUPS):
                vals = plsc.load_gather(
                    bufs.at[slot], [rows_l[g], colb_l[g] + cperm]
                )
                plsc.store_scatter(
                    slabs.at[slot], [cperm, rows_l[g]], vals
                )

    prep(0, 0)
    start_gather(0)

    @pl.loop(0, SEQ, step=2)
    def _grp(s0):
        for inner in range(2):
            s = s0 + inner
            slot = inner
            nxt = 1 - inner

            @pl.when(s + 1 < SEQ)
            def _():
                prep(s + 1, nxt)
                start_gather(nxt)

            wait_gather(slot)

            @pl.when(s >= 2)
            def _():
                wait_write(s - 2, slot)

            transpose_select(slot)
            start_write(s, slot)

    wait_write(SEQ - 2, 0)
    wait_write(SEQ - 1, 1)


def kernel(input, weight):
    idx_t = input.T
    table = weight.reshape(NUM_ROWS // 2, 2 * DIM)
    out = _emb_kernel(idx_t, table)
    return out.transpose(2, 0, 1)


# speedup vs baseline: 1.2884x; 1.0008x over previous
import functools

import jax
import jax.numpy as jnp
from jax import lax
from jax.experimental import pallas as pl
from jax.experimental.pallas import tpu as pltpu
from jax.experimental.pallas import tpu_sc as plsc

NUM_ROWS = 1000000
DIM = 64
BATCH = 4096
SEQ = 50

_info = plsc.get_sparse_core_info()
NC, NS, NL = _info.num_cores, _info.num_subcores, _info.num_lanes
NW = NC * NS

CHUNK = 128
GROUPS = CHUNK // NL

_mesh = plsc.VectorSubcoreMesh(core_axis_name="c", subcore_axis_name="s")


@functools.partial(
    pl.kernel,
    out_type=jax.ShapeDtypeStruct((SEQ, DIM, BATCH), jnp.float32),
    mesh=_mesh,
    scratch_types=[
        pltpu.VMEM((SEQ, CHUNK), jnp.int32),
        pltpu.VMEM((2, CHUNK), jnp.int32),
        pltpu.VMEM((2, CHUNK), jnp.int32),
        pltpu.VMEM((2, CHUNK, CHUNK), jnp.float32),
        pltpu.VMEM((2, DIM, CHUNK), jnp.float32),
        pltpu.SemaphoreType.DMA((2,)),
        pltpu.SemaphoreType.DMA((2,)),
    ],
    compiler_params=pltpu.CompilerParams(needs_layout_passes=False),
)
def _emb_kernel(idx_hbm, table_hbm, out_hbm, idx_v, pair_v, colb_v, bufs, slabs,
                gsem, wsem):
    wid = lax.axis_index("s") * NC + lax.axis_index("c")
    b0 = wid * CHUNK
    pltpu.sync_copy(idx_hbm.at[:, pl.ds(b0, CHUNK)], idx_v)

    def prep(s, slot):
        for g in range(GROUPS):
            raw = idx_v[s, pl.ds(g * NL, NL)]
            pair_v[slot, pl.ds(g * NL, NL)] = lax.shift_right_logical(raw, 1)
            colb_v[slot, pl.ds(g * NL, NL)] = (raw & 1) * DIM

    def start_gather(slot):
        pltpu.async_copy(table_hbm.at[pair_v.at[slot]], bufs.at[slot],
                         gsem.at[slot])

    def wait_gather(slot):
        pltpu.make_async_copy(table_hbm.at[pair_v.at[slot]], bufs.at[slot],
                              gsem.at[slot]).wait()

    def start_write(s, slot):
        pltpu.async_copy(slabs.at[slot],
                         out_hbm.at[s, :, pl.ds(b0, CHUNK)], wsem.at[slot])

    def wait_write(s, slot):
        pltpu.make_async_copy(slabs.at[slot],
                              out_hbm.at[s, :, pl.ds(b0, CHUNK)],
                              wsem.at[slot]).wait()

    def transpose_select(slot):
        iota = jax.lax.broadcasted_iota(jnp.int32, (NL,), 0)
        rows_l = [iota + g * NL for g in range(GROUPS)]
        colb_l = [colb_v[slot, pl.ds(g * NL, NL)] for g in range(GROUPS)]

        @pl.loop(0, DIM, unroll=8)
        def _c(c):
            cperm = (iota + c) & (DIM - 1)
            for g in range(GROUPS):
                vals = plsc.load_gather(
                    bufs.at[slot], [rows_l[g], colb_l[g] + cperm]
                )
                plsc.store_scatter(
                    slabs.at[slot], [cperm, rows_l[g]], vals
                )

    prep(0, 0)
    start_gather(0)

    @pl.loop(0, SEQ, step=2)
    def _grp(s0):
        for inner in range(2):
            s = s0 + inner
            slot = inner
            nxt = 1 - inner

            @pl.when(s + 1 < SEQ)
            def _():
                prep(s + 1, nxt)
                start_gather(nxt)

            wait_gather(slot)

            @pl.when(s >= 2)
            def _():
                wait_write(s - 2, slot)

            transpose_select(slot)
            start_write(s, slot)

    wait_write(SEQ - 2, 0)
    wait_write(SEQ - 1, 1)


def kernel(input, weight):
    idx_t = input.T
    table = weight.reshape(NUM_ROWS // 2, 2 * DIM)
    out = _emb_kernel(idx_t, table)
    return out.transpose(2, 0, 1)
